# baseline (speedup 1.0000x reference)
import functools

import jax
import jax.numpy as jnp
from jax import lax
from jax.experimental import pallas as pl
from jax.experimental.pallas import tpu as pltpu
from jax.experimental.pallas import tpu_sc as plsc

DIN = 768
NFEAT = 16384
KTOP = 32
NTOK = 2048

RBLK = 128
FT_ENC = 1024
NCHUNK = 32
CHUNK = NFEAT // NCHUNK
NCAND = 2048

NC = 2
NS = 16
NW = NC * NS
ROWS_PER_W = NTOK // NW


def _encode_kernel(x_ref, we_ref, be_ref, z_ref, m_ref):
    f = pl.program_id(1)
    nf = pl.num_programs(1)
    z_part = (
        jnp.dot(x_ref[...], we_ref[...], preferred_element_type=jnp.float32)
        + be_ref[...]
    )
    z_ref[:, pl.ds(f * FT_ENC, FT_ENC)] = z_part

    @pl.when(f == nf - 1)
    def _chunk_maxes():
        z = z_ref[...].reshape(RBLK, NCHUNK, CHUNK)
        m_ref[...] = jnp.max(z, axis=2)


def _sc_decode_body(z_hbm, m_hbm, wd_hbm, bd_hbm, out_hbm,
                    z_v, m_v, cand_v, ci_v, vals_v, idx_v, rows_v,
                    rec_v, bd_v, tmpf_v, ni_v, sem):
    wid = lax.axis_index("s") * NC + lax.axis_index("c")
    base = wid * ROWS_PER_W
    pltpu.sync_copy(bd_hbm, bd_v)
    lanes = lax.broadcasted_iota(jnp.int32, (16,), 0)
    neg = jnp.full((16,), -jnp.inf, jnp.float32)
    def _splat_max(v):
        r = v
        for k in (1, 2, 4, 8):
            tmpf_v[pl.ds(0, 16)] = r
            r = jnp.maximum(r, plsc.load_gather(tmpf_v, [lanes ^ k]))
        return r

    def row_body(rr, carry):
        r = base + rr
        pltpu.sync_copy(z_hbm.at[r], z_v)
        pltpu.sync_copy(m_hbm.at[r], m_v)
        mneg = jnp.maximum(-m_v[pl.ds(0, 16)], -m_v[pl.ds(16, 16)])
        t0v = -_splat_max(mneg)

        def pre(i, c):
            cand_v[pl.ds(i * 16, 16)] = neg
            return c
        lax.fori_loop(0, NCAND // 16, pre, 0)

        def _prefix(m32):
            cs = m32
            for k in (1, 2, 4, 8):
                ni_v[pl.ds(0, 16)] = cs
                g = plsc.load_gather(ni_v, [jnp.maximum(lanes - k, 0)])
                cs = cs + jnp.where(lanes >= k, g, 0)
            return cs

        def scan(i, cnt):
            v = z_v[pl.ds(i * 16, 16)]
            msk = v >= t0v
            ns = plsc.all_reduce_population_count(msk)[0]

            def compact(c):
                cs = _prefix(jnp.where(msk, 1, 0).astype(jnp.int32))
                pos = jnp.minimum(c + cs - 1, NCAND - 1)
                plsc.store_scatter(cand_v, [pos], v, mask=msk)
                col = (i * 16 + lanes).astype(jnp.int32)
                plsc.store_scatter(ci_v, [pos], col, mask=msk)
                return c + ns

            return lax.cond(ns > 0, compact, lambda c: c, cnt)

        n = lax.fori_loop(0, NFEAT // 16, scan, jnp.int32(0))
        nv = (n + 15) // 16

        def sel(j, m):
            def sweep(i, acc):
                v = cand_v[pl.ds(i * 16, 16)]
                return jnp.maximum(acc, jnp.where(v < m, v, neg))
            acc = lax.fori_loop(0, nv, sweep, neg)
            return _splat_max(acc)

        tv = lax.fori_loop(0, KTOP, sel,
                           jnp.full((16,), jnp.inf, jnp.float32))

        def pick(i, cnt):
            v = cand_v[pl.ds(i * 16, 16)]
            ii = ci_v[pl.ds(i * 16, 16)]
            msk = v >= tv
            ns = plsc.all_reduce_population_count(msk)[0]

            def compact(c):
                cs = _prefix(jnp.where(msk, 1, 0).astype(jnp.int32))
                pos = jnp.minimum(c + cs - 1, KTOP - 1)
                plsc.store_scatter(vals_v, [pos], v, mask=msk)
                plsc.store_scatter(idx_v, [pos], ii, mask=msk)
                return c + ns

            return lax.cond(ns > 0, compact, lambda c: c, cnt)

        lax.fori_loop(0, nv, pick, jnp.int32(0))

        pltpu.async_copy(wd_hbm.at[idx_v], rows_v, sem).wait()

        vlo = vals_v[pl.ds(0, 16)]
        vhi = vals_v[pl.ds(16, 16)]
        vb = [jnp.maximum(jnp.broadcast_to(vlo[j], (16,)), 0.0)
              for j in range(16)]
        vb += [jnp.maximum(jnp.broadcast_to(vhi[j], (16,)), 0.0)
               for j in range(16)]

        def mac(c, carry2):
            a0 = bd_v[pl.ds(c * 16, 16)]
            a1 = jnp.zeros((16,), jnp.float32)
            a2 = jnp.zeros((16,), jnp.float32)
            a3 = jnp.zeros((16,), jnp.float32)
            for j in range(0, KTOP, 4):
                a0 = a0 + rows_v[j, pl.ds(c * 16, 16)] * vb[j]
                a1 = a1 + rows_v[j + 1, pl.ds(c * 16, 16)] * vb[j + 1]
                a2 = a2 + rows_v[j + 2, pl.ds(c * 16, 16)] * vb[j + 2]
                a3 = a3 + rows_v[j + 3, pl.ds(c * 16, 16)] * vb[j + 3]
            rec_v[pl.ds(c * 16, 16)] = (a0 + a1) + (a2 + a3)
            return carry2
        lax.fori_loop(0, DIN // 16, mac, 0)

        pltpu.sync_copy(rec_v, out_hbm.at[r])
        return carry

    lax.fori_loop(0, ROWS_PER_W, row_body, 0)


def kernel(x, W_enc, b_enc, W_dec, b_dec):
    z, m = pl.pallas_call(
        _encode_kernel,
        grid=(NTOK // RBLK, NFEAT // FT_ENC),
        in_specs=[
            pl.BlockSpec((RBLK, DIN), lambda t, f: (t, 0)),
            pl.BlockSpec((DIN, FT_ENC), lambda t, f: (0, f)),
            pl.BlockSpec((1, FT_ENC), lambda t, f: (0, f)),
        ],
        out_specs=[
            pl.BlockSpec((RBLK, NFEAT), lambda t, f: (t, 0)),
            pl.BlockSpec((RBLK, NCHUNK), lambda t, f: (t, 0)),
        ],
        out_shape=[
            jax.ShapeDtypeStruct((NTOK, NFEAT), jnp.float32),
            jax.ShapeDtypeStruct((NTOK, NCHUNK), jnp.float32),
        ],
        compiler_params=pltpu.CompilerParams(
            dimension_semantics=("parallel", "arbitrary"),
        ),
    )(x, W_enc, b_enc.reshape(1, NFEAT))

    sc_call = functools.partial(
        pl.kernel,
        out_type=jax.ShapeDtypeStruct((NTOK, DIN), jnp.float32),
        mesh=plsc.VectorSubcoreMesh(core_axis_name="c", subcore_axis_name="s"),
        compiler_params=pltpu.CompilerParams(needs_layout_passes=False),
        scratch_types=[
            pltpu.VMEM((NFEAT,), jnp.float32),
            pltpu.VMEM((NCHUNK,), jnp.float32),
            pltpu.VMEM((NCAND,), jnp.float32),
            pltpu.VMEM((NCAND,), jnp.int32),
            pltpu.VMEM((KTOP,), jnp.float32),
            pltpu.VMEM((KTOP,), jnp.int32),
            pltpu.VMEM((KTOP, DIN), jnp.float32),
            pltpu.VMEM((DIN,), jnp.float32),
            pltpu.VMEM((DIN,), jnp.float32),
            pltpu.VMEM((16,), jnp.float32),
            pltpu.VMEM((16,), jnp.int32),
            pltpu.SemaphoreType.DMA,
        ],
    )(_sc_decode_body)

    recon = sc_call(z, m, W_dec, b_dec)
    return recon


# speedup vs baseline: 1.0230x; 1.0230x over previous
import functools

import jax
import jax.numpy as jnp
from jax import lax
from jax.experimental import pallas as pl
from jax.experimental.pallas import tpu as pltpu
from jax.experimental.pallas import tpu_sc as plsc

DIN = 768
NFEAT = 16384
KTOP = 32
NTOK = 2048

RBLK = 128
FT_ENC = 1024
NCHUNK = 32
CHUNK = NFEAT // NCHUNK
NCAND = 2048

NC = 2
NS = 16
NW = NC * NS
ROWS_PER_W = NTOK // NW


def _encode_kernel(x_ref, we_ref, be_ref, z_ref, m_ref):
    f = pl.program_id(1)
    nf = pl.num_programs(1)
    z_part = (
        jnp.dot(x_ref[...], we_ref[...], preferred_element_type=jnp.float32)
        + be_ref[...]
    )
    z_ref[:, pl.ds(f * FT_ENC, FT_ENC)] = z_part

    @pl.when(f == nf - 1)
    def _chunk_maxes():
        z = z_ref[...].reshape(RBLK, NCHUNK, CHUNK)
        m_ref[...] = jnp.max(z, axis=2)


def _sc_decode_body(z_hbm, m_hbm, wd_hbm, bd_hbm, out_hbm,
                    z_v, z2_v, m_v, cand_v, ci_v, vals_v, idx_v, rows_v,
                    rec_v, bd_v, tmpf_v, ni_v, semz, semz2, semg):
    wid = lax.axis_index("s") * NC + lax.axis_index("c")
    base = wid * ROWS_PER_W
    pltpu.sync_copy(bd_hbm, bd_v)
    lanes = lax.broadcasted_iota(jnp.int32, (16,), 0)
    neg = jnp.full((16,), -jnp.inf, jnp.float32)
    def _splat_max(v):
        r = v
        for k in (1, 2, 4, 8):
            tmpf_v[pl.ds(0, 16)] = r
            r = jnp.maximum(r, plsc.load_gather(tmpf_v, [lanes ^ k]))
        return r

    def _process(r, zbuf, semx, rnext):
        pltpu.make_async_copy(z_hbm.at[r], zbuf, semx).wait()
        pltpu.sync_copy(m_hbm.at[r], m_v)
        mneg = jnp.maximum(-m_v[pl.ds(0, 16)], -m_v[pl.ds(16, 16)])
        t0v = -_splat_max(mneg)

        def pre(i, c):
            cand_v[pl.ds(i * 16, 16)] = neg
            return c
        lax.fori_loop(0, NCAND // 16, pre, 0)

        def _prefix(m32):
            cs = m32
            for k in (1, 2, 4, 8):
                ni_v[pl.ds(0, 16)] = cs
                g = plsc.load_gather(ni_v, [jnp.maximum(lanes - k, 0)])
                cs = cs + jnp.where(lanes >= k, g, 0)
            return cs

        def scan(i, cnt):
            v = zbuf[pl.ds(i * 16, 16)]
            msk = v >= t0v
            ns = plsc.all_reduce_population_count(msk)[0]

            def compact(c):
                cs = _prefix(jnp.where(msk, 1, 0).astype(jnp.int32))
                pos = jnp.minimum(c + cs - 1, NCAND - 1)
                plsc.store_scatter(cand_v, [pos], v, mask=msk)
                col = (i * 16 + lanes).astype(jnp.int32)
                plsc.store_scatter(ci_v, [pos], col, mask=msk)
                return c + ns

            return lax.cond(ns > 0, compact, lambda c: c, cnt)

        n = lax.fori_loop(0, NFEAT // 16, scan, jnp.int32(0))
        nv = (n + 15) // 16

        @pl.when(rnext < base + ROWS_PER_W)
        def _refill():
            pltpu.async_copy(z_hbm.at[rnext], zbuf, semx)

        def sel(j, m):
            def sweep(i, acc):
                v = cand_v[pl.ds(i * 16, 16)]
                return jnp.maximum(acc, jnp.where(v < m, v, neg))
            acc = lax.fori_loop(0, nv, sweep, neg)
            return _splat_max(acc)

        tv = lax.fori_loop(0, KTOP, sel,
                           jnp.full((16,), jnp.inf, jnp.float32))

        def pick(i, cnt):
            v = cand_v[pl.ds(i * 16, 16)]
            ii = ci_v[pl.ds(i * 16, 16)]
            msk = v >= tv
            ns = plsc.all_reduce_population_count(msk)[0]

            def compact(c):
                cs = _prefix(jnp.where(msk, 1, 0).astype(jnp.int32))
                pos = jnp.minimum(c + cs - 1, KTOP - 1)
                plsc.store_scatter(vals_v, [pos], v, mask=msk)
                plsc.store_scatter(idx_v, [pos], ii, mask=msk)
                return c + ns

            return lax.cond(ns > 0, compact, lambda c: c, cnt)

        lax.fori_loop(0, nv, pick, jnp.int32(0))

        pltpu.async_copy(wd_hbm.at[idx_v], rows_v, semg).wait()

        vlo = vals_v[pl.ds(0, 16)]
        vhi = vals_v[pl.ds(16, 16)]
        vb = [jnp.maximum(jnp.broadcast_to(vlo[j], (16,)), 0.0)
              for j in range(16)]
        vb += [jnp.maximum(jnp.broadcast_to(vhi[j], (16,)), 0.0)
               for j in range(16)]

        def mac(c, carry2):
            a0 = bd_v[pl.ds(c * 16, 16)]
            a1 = jnp.zeros((16,), jnp.float32)
            a2 = jnp.zeros((16,), jnp.float32)
            a3 = jnp.zeros((16,), jnp.float32)
            for j in range(0, KTOP, 4):
                a0 = a0 + rows_v[j, pl.ds(c * 16, 16)] * vb[j]
                a1 = a1 + rows_v[j + 1, pl.ds(c * 16, 16)] * vb[j + 1]
                a2 = a2 + rows_v[j + 2, pl.ds(c * 16, 16)] * vb[j + 2]
                a3 = a3 + rows_v[j + 3, pl.ds(c * 16, 16)] * vb[j + 3]
            rec_v[pl.ds(c * 16, 16)] = (a0 + a1) + (a2 + a3)
            return carry2
        lax.fori_loop(0, DIN // 16, mac, 0)

        pltpu.sync_copy(rec_v, out_hbm.at[r])

    pltpu.async_copy(z_hbm.at[base], z_v, semz)
    pltpu.async_copy(z_hbm.at[base + 1], z2_v, semz2)

    def pair_body(p, carry):
        r0 = base + 2 * p
        _process(r0, z_v, semz, r0 + 2)
        _process(r0 + 1, z2_v, semz2, r0 + 3)
        return carry

    lax.fori_loop(0, ROWS_PER_W // 2, pair_body, 0)


def kernel(x, W_enc, b_enc, W_dec, b_dec):
    z, m = pl.pallas_call(
        _encode_kernel,
        grid=(NTOK // RBLK, NFEAT // FT_ENC),
        in_specs=[
            pl.BlockSpec((RBLK, DIN), lambda t, f: (t, 0)),
            pl.BlockSpec((DIN, FT_ENC), lambda t, f: (0, f)),
            pl.BlockSpec((1, FT_ENC), lambda t, f: (0, f)),
        ],
        out_specs=[
            pl.BlockSpec((RBLK, NFEAT), lambda t, f: (t, 0)),
            pl.BlockSpec((RBLK, NCHUNK), lambda t, f: (t, 0)),
        ],
        out_shape=[
            jax.ShapeDtypeStruct((NTOK, NFEAT), jnp.float32),
            jax.ShapeDtypeStruct((NTOK, NCHUNK), jnp.float32),
        ],
        compiler_params=pltpu.CompilerParams(
            dimension_semantics=("parallel", "arbitrary"),
        ),
    )(x, W_enc, b_enc.reshape(1, NFEAT))

    sc_call = functools.partial(
        pl.kernel,
        out_type=jax.ShapeDtypeStruct((NTOK, DIN), jnp.float32),
        mesh=plsc.VectorSubcoreMesh(core_axis_name="c", subcore_axis_name="s"),
        compiler_params=pltpu.CompilerParams(needs_layout_passes=False),
        scratch_types=[
            pltpu.VMEM((NFEAT,), jnp.float32),
            pltpu.VMEM((NFEAT,), jnp.float32),
            pltpu.VMEM((NCHUNK,), jnp.float32),
            pltpu.VMEM((NCAND,), jnp.float32),
            pltpu.VMEM((NCAND,), jnp.int32),
            pltpu.VMEM((KTOP,), jnp.float32),
            pltpu.VMEM((KTOP,), jnp.int32),
            pltpu.VMEM((KTOP, DIN), jnp.float32),
            pltpu.VMEM((DIN,), jnp.float32),
            pltpu.VMEM((DIN,), jnp.float32),
            pltpu.VMEM((16,), jnp.float32),
            pltpu.VMEM((16,), jnp.int32),
            pltpu.SemaphoreType.DMA,
            pltpu.SemaphoreType.DMA,
            pltpu.SemaphoreType.DMA,
        ],
    )(_sc_decode_body)

    recon = sc_call(z, m, W_dec, b_dec)
    return recon


# speedup vs baseline: 1.1033x; 1.0785x over previous
import functools

import jax
import jax.numpy as jnp
from jax import lax
from jax.experimental import pallas as pl
from jax.experimental.pallas import tpu as pltpu
from jax.experimental.pallas import tpu_sc as plsc

DIN = 768
NFEAT = 16384
KTOP = 32
NTOK = 2048

RBLK = 128
FT_ENC = 1024
NCHUNK = 32
CHUNK = NFEAT // NCHUNK
NCAND = 2048

NC = 2
NS = 16
NW = NC * NS
ROWS_PER_W = NTOK // NW


def _encode_kernel(x_ref, we_ref, be_ref, z_ref, m_ref):
    f = pl.program_id(1)
    nf = pl.num_programs(1)
    z_part = (
        jnp.dot(x_ref[...], we_ref[...], preferred_element_type=jnp.float32)
        + be_ref[...]
    )
    z_ref[:, pl.ds(f * FT_ENC, FT_ENC)] = z_part

    @pl.when(f == nf - 1)
    def _chunk_maxes():
        z = z_ref[...].reshape(RBLK, NCHUNK, CHUNK)
        m_ref[...] = jnp.max(z, axis=2)


def _sc_decode_body(z_hbm, m_hbm, wd_hbm, bd_hbm, out_hbm,
                    z_v, z2_v, m_v, cand_v, ci_v, vals_v, idx_v, rows_v,
                    rec_v, bd_v, tmpf_v, ni_v, semz, semz2, semg):
    wid = lax.axis_index("s") * NC + lax.axis_index("c")
    base = wid * ROWS_PER_W
    pltpu.sync_copy(bd_hbm, bd_v)
    lanes = lax.broadcasted_iota(jnp.int32, (16,), 0)
    neg = jnp.full((16,), -jnp.inf, jnp.float32)
    def _splat_max(v):
        r = v
        for k in (1, 2, 4, 8):
            tmpf_v[pl.ds(0, 16)] = r
            r = jnp.maximum(r, plsc.load_gather(tmpf_v, [lanes ^ k]))
        return r

    def _process(r, zbuf, semx, rnext):
        pltpu.make_async_copy(z_hbm.at[r], zbuf, semx).wait()
        pltpu.sync_copy(m_hbm.at[r], m_v)
        mneg = jnp.maximum(-m_v[pl.ds(0, 16)], -m_v[pl.ds(16, 16)])
        t0v = -_splat_max(mneg)

        def pre(i, c):
            cand_v[pl.ds(i * 16, 16)] = neg
            return c
        lax.fori_loop(0, NCAND // 16, pre, 0, unroll=8)

        def _prefix(m32):
            cs = m32
            for k in (1, 2, 4, 8):
                ni_v[pl.ds(0, 16)] = cs
                g = plsc.load_gather(ni_v, [jnp.maximum(lanes - k, 0)])
                cs = cs + jnp.where(lanes >= k, g, 0)
            return cs

        def scan(i, cnt):
            v = zbuf[pl.ds(i * 16, 16)]
            msk = v >= t0v
            ns = plsc.all_reduce_population_count(msk)[0]

            def compact(c):
                cs = _prefix(jnp.where(msk, 1, 0).astype(jnp.int32))
                pos = jnp.minimum(c + cs - 1, NCAND - 1)
                plsc.store_scatter(cand_v, [pos], v, mask=msk)
                col = (i * 16 + lanes).astype(jnp.int32)
                plsc.store_scatter(ci_v, [pos], col, mask=msk)
                return c + ns

            return lax.cond(ns > 0, compact, lambda c: c, cnt)

        n = lax.fori_loop(0, NFEAT // 16, scan, jnp.int32(0), unroll=8)
        nv = (n + 15) // 16

        @pl.when(rnext < base + ROWS_PER_W)
        def _refill():
            pltpu.async_copy(z_hbm.at[rnext], zbuf, semx)

        def sel(j, m):
            def sweep(i, acc):
                v = cand_v[pl.ds(i * 16, 16)]
                return jnp.maximum(acc, jnp.where(v < m, v, neg))
            acc = lax.fori_loop(0, nv, sweep, neg)
            return _splat_max(acc)

        tv = lax.fori_loop(0, KTOP, sel,
                           jnp.full((16,), jnp.inf, jnp.float32))

        def pick(i, cnt):
            v = cand_v[pl.ds(i * 16, 16)]
            ii = ci_v[pl.ds(i * 16, 16)]
            msk = v >= tv
            ns = plsc.all_reduce_population_count(msk)[0]

            def compact(c):
                cs = _prefix(jnp.where(msk, 1, 0).astype(jnp.int32))
                pos = jnp.minimum(c + cs - 1, KTOP - 1)
                plsc.store_scatter(vals_v, [pos], v, mask=msk)
                plsc.store_scatter(idx_v, [pos], ii, mask=msk)
                return c + ns

            return lax.cond(ns > 0, compact, lambda c: c, cnt)

        lax.fori_loop(0, nv, pick, jnp.int32(0))

        pltpu.async_copy(wd_hbm.at[idx_v], rows_v, semg).wait()

        vlo = vals_v[pl.ds(0, 16)]
        vhi = vals_v[pl.ds(16, 16)]
        vb = [jnp.maximum(jnp.broadcast_to(vlo[j], (16,)), 0.0)
              for j in range(16)]
        vb += [jnp.maximum(jnp.broadcast_to(vhi[j], (16,)), 0.0)
               for j in range(16)]

        def mac(c, carry2):
            a0 = bd_v[pl.ds(c * 16, 16)]
            a1 = jnp.zeros((16,), jnp.float32)
            a2 = jnp.zeros((16,), jnp.float32)
            a3 = jnp.zeros((16,), jnp.float32)
            for j in range(0, KTOP, 4):
                a0 = a0 + rows_v[j, pl.ds(c * 16, 16)] * vb[j]
                a1 = a1 + rows_v[j + 1, pl.ds(c * 16, 16)] * vb[j + 1]
                a2 = a2 + rows_v[j + 2, pl.ds(c * 16, 16)] * vb[j + 2]
                a3 = a3 + rows_v[j + 3, pl.ds(c * 16, 16)] * vb[j + 3]
            rec_v[pl.ds(c * 16, 16)] = (a0 + a1) + (a2 + a3)
            return carry2
        lax.fori_loop(0, DIN // 16, mac, 0, unroll=2)

        pltpu.sync_copy(rec_v, out_hbm.at[r])

    pltpu.async_copy(z_hbm.at[base], z_v, semz)
    pltpu.async_copy(z_hbm.at[base + 1], z2_v, semz2)

    def pair_body(p, carry):
        r0 = base + 2 * p
        _process(r0, z_v, semz, r0 + 2)
        _process(r0 + 1, z2_v, semz2, r0 + 3)
        return carry

    lax.fori_loop(0, ROWS_PER_W // 2, pair_body, 0)


def kernel(x, W_enc, b_enc, W_dec, b_dec):
    z, m = pl.pallas_call(
        _encode_kernel,
        grid=(NTOK // RBLK, NFEAT // FT_ENC),
        in_specs=[
            pl.BlockSpec((RBLK, DIN), lambda t, f: (t, 0)),
            pl.BlockSpec((DIN, FT_ENC), lambda t, f: (0, f)),
            pl.BlockSpec((1, FT_ENC), lambda t, f: (0, f)),
        ],
        out_specs=[
            pl.BlockSpec((RBLK, NFEAT), lambda t, f: (t, 0)),
            pl.BlockSpec((RBLK, NCHUNK), lambda t, f: (t, 0)),
        ],
        out_shape=[
            jax.ShapeDtypeStruct((NTOK, NFEAT), jnp.float32),
            jax.ShapeDtypeStruct((NTOK, NCHUNK), jnp.float32),
        ],
        compiler_params=pltpu.CompilerParams(
            dimension_semantics=("parallel", "arbitrary"),
        ),
    )(x, W_enc, b_enc.reshape(1, NFEAT))

    sc_call = functools.partial(
        pl.kernel,
        out_type=jax.ShapeDtypeStruct((NTOK, DIN), jnp.float32),
        mesh=plsc.VectorSubcoreMesh(core_axis_name="c", subcore_axis_name="s"),
        compiler_params=pltpu.CompilerParams(needs_layout_passes=False),
        scratch_types=[
            pltpu.VMEM((NFEAT,), jnp.float32),
            pltpu.VMEM((NFEAT,), jnp.float32),
            pltpu.VMEM((NCHUNK,), jnp.float32),
            pltpu.VMEM((NCAND,), jnp.float32),
            pltpu.VMEM((NCAND,), jnp.int32),
            pltpu.VMEM((KTOP,), jnp.float32),
            pltpu.VMEM((KTOP,), jnp.int32),
            pltpu.VMEM((KTOP, DIN), jnp.float32),
            pltpu.VMEM((DIN,), jnp.float32),
            pltpu.VMEM((DIN,), jnp.float32),
            pltpu.VMEM((16,), jnp.float32),
            pltpu.VMEM((16,), jnp.int32),
            pltpu.SemaphoreType.DMA,
            pltpu.SemaphoreType.DMA,
            pltpu.SemaphoreType.DMA,
        ],
    )(_sc_decode_body)

    recon = sc_call(z, m, W_dec, b_dec)
    return recon
